# baseline (speedup 1.0000x reference)
import jax
import jax.numpy as jnp
from jax import lax
from jax.experimental import pallas as pl
from jax.experimental.pallas import tpu as pltpu
from jax.experimental.pallas import tpu_sc as plsc

_B = 16384
_D = 32
_DP = _D // 2
_V = 1000000
_W = 128
_QD = 7816
_VP = _QD * _W
_CB = 496 * _W
_NSTEP = -(-_QD // 496)
_NC = 2
_NS = 16
_NW = _NC * _NS
_BPW = _B // _NW
_CH = 128
_NCH = _BPW // _CH
_L = 16


def _pack_body(zt_ref, out_ref):
    x = zt_ref[...]
    a = lax.bitcast_convert_type(
        x[:_DP].astype(jnp.bfloat16), jnp.uint16).astype(jnp.uint32)
    b = lax.bitcast_convert_type(
        x[_DP:].astype(jnp.bfloat16), jnp.uint16).astype(jnp.uint32)
    u = a | (b << 16)
    out_ref[...] = u.reshape(_DP, _CB // _W, _W)


def _pack(zt):
    return pl.pallas_call(
        _pack_body,
        grid=(_NSTEP,),
        in_specs=[pl.BlockSpec((_D, _CB), lambda j: (0, j))],
        out_specs=pl.BlockSpec((_DP, _CB // _W, _W), lambda j: (0, j, 0)),
        out_shape=jax.ShapeDtypeStruct((_DP, _QD, _W), jnp.uint32),
    )(zt)


def _body(zf_hbm, rows_hbm, cols_hbm, out_hbm, ridx, cidx, gidx, rbuf, cbuf,
          outv, sem):
    wid = lax.axis_index("s") * _NC + lax.axis_index("c")
    base = wid * _BPW

    for j in range(_NCH):
        pltpu.sync_copy(rows_hbm.at[pl.ds(base + j * _CH, _CH)], ridx.at[j])
        pltpu.sync_copy(cols_hbm.at[pl.ds(base + j * _CH, _CH)], cidx.at[j])

    copies = []
    for d in range(_DP):
        for j in range(_NCH):
            for k in range(_CH // _L):
                s = pl.ds(k * _L, _L)
                gidx[2 * d, j, s] = ridx[j, s] + d * _VP
                gidx[2 * d + 1, j, s] = cidx[j, s] + d * _VP
        for j in range(_NCH):
            copies.append(pltpu.async_copy(
                zf_hbm.at[gidx.at[2 * d, j]],
                rbuf.at[d, pl.ds(j * _CH, _CH)], sem))
            copies.append(pltpu.async_copy(
                zf_hbm.at[gidx.at[2 * d + 1, j]],
                cbuf.at[d, pl.ds(j * _CH, _CH)], sem))
    for cp in copies:
        cp.wait()

    def group(g, carry):
        s = pl.ds(g * _L, _L)
        acc = jnp.zeros((_L,), jnp.float32)
        for d in range(_DP):
            ru = rbuf[d, s]
            cu = cbuf[d, s]
            rlo = plsc.bitcast(ru << 16, jnp.float32)
            clo = plsc.bitcast(cu << 16, jnp.float32)
            hi = jnp.uint32(0xFFFF0000)
            rhi = plsc.bitcast(ru & hi, jnp.float32)
            chi = plsc.bitcast(cu & hi, jnp.float32)
            acc = acc + rlo * clo + rhi * chi
        outv[s] = acc
        return carry

    lax.fori_loop(0, _BPW // _L, group, 0)
    pltpu.sync_copy(outv, out_hbm.at[pl.ds(base, _BPW)])


def kernel(z, drp_rows, drp_cols):
    zf = _pack(z.T).reshape(-1)
    mesh = plsc.VectorSubcoreMesh(core_axis_name="c", subcore_axis_name="s")
    f = pl.kernel(
        _body,
        out_type=jax.ShapeDtypeStruct((_B,), jnp.float32),
        mesh=mesh,
        compiler_params=pltpu.CompilerParams(
            needs_layout_passes=False, use_tc_tiling_on_sc=True),
        scratch_types=[
            pltpu.VMEM((_NCH, _CH), jnp.int32),
            pltpu.VMEM((_NCH, _CH), jnp.int32),
            pltpu.VMEM((2 * _DP, _NCH, _CH), jnp.int32),
            pltpu.VMEM((_DP, _BPW), jnp.uint32),
            pltpu.VMEM((_DP, _BPW), jnp.uint32),
            pltpu.VMEM((_BPW,), jnp.float32),
            pltpu.SemaphoreType.DMA,
        ],
    )
    return f(zf, drp_rows.astype(jnp.int32), drp_cols.astype(jnp.int32))


# speedup vs baseline: 1.0132x; 1.0132x over previous
import jax
import jax.numpy as jnp
from jax import lax
from jax.experimental import pallas as pl
from jax.experimental.pallas import tpu as pltpu
from jax.experimental.pallas import tpu_sc as plsc

_B = 16384
_D = 32
_DP = _D // 2
_V = 1000000
_W = 128
_QD = 7816
_VP = _QD * _W
_CB = 248 * _W
_NSTEP = -(-_QD // 248)
_NC = 2
_NS = 16
_NW = _NC * _NS
_BPW = _B // _NW
_CH = 128
_NCH = _BPW // _CH
_L = 16


def _pack_body(zt_ref, out_ref):
    x = zt_ref[...]
    a = lax.bitcast_convert_type(
        x[:_DP].astype(jnp.bfloat16), jnp.uint16).astype(jnp.uint32)
    b = lax.bitcast_convert_type(
        x[_DP:].astype(jnp.bfloat16), jnp.uint16).astype(jnp.uint32)
    u = a | (b << 16)
    out_ref[...] = u.reshape(_DP, _CB // _W, _W)


def _pack(zt):
    return pl.pallas_call(
        _pack_body,
        grid=(_NSTEP,),
        in_specs=[pl.BlockSpec((_D, _CB), lambda j: (0, j))],
        out_specs=pl.BlockSpec((_DP, _CB // _W, _W), lambda j: (0, j, 0)),
        out_shape=jax.ShapeDtypeStruct((_DP, _QD, _W), jnp.uint32),
    )(zt)


def _body(zf_hbm, rows_hbm, cols_hbm, out_hbm, ridx, cidx, gidx, rbuf, cbuf,
          outv, sem):
    wid = lax.axis_index("s") * _NC + lax.axis_index("c")
    base = wid * _BPW

    for j in range(_NCH):
        pltpu.sync_copy(rows_hbm.at[pl.ds(base + j * _CH, _CH)], ridx.at[j])
        pltpu.sync_copy(cols_hbm.at[pl.ds(base + j * _CH, _CH)], cidx.at[j])

    copies = []
    for d in range(_DP):
        for j in range(_NCH):
            for k in range(_CH // _L):
                s = pl.ds(k * _L, _L)
                gidx[2 * d, j, s] = ridx[j, s] + d * _VP
                gidx[2 * d + 1, j, s] = cidx[j, s] + d * _VP
        for j in range(_NCH):
            copies.append(pltpu.async_copy(
                zf_hbm.at[gidx.at[2 * d, j]],
                rbuf.at[d, pl.ds(j * _CH, _CH)], sem))
            copies.append(pltpu.async_copy(
                zf_hbm.at[gidx.at[2 * d + 1, j]],
                cbuf.at[d, pl.ds(j * _CH, _CH)], sem))
    for cp in copies:
        cp.wait()

    def group(g, carry):
        s = pl.ds(g * _L, _L)
        acc = jnp.zeros((_L,), jnp.float32)
        for d in range(_DP):
            ru = rbuf[d, s]
            cu = cbuf[d, s]
            rlo = plsc.bitcast(ru << 16, jnp.float32)
            clo = plsc.bitcast(cu << 16, jnp.float32)
            hi = jnp.uint32(0xFFFF0000)
            rhi = plsc.bitcast(ru & hi, jnp.float32)
            chi = plsc.bitcast(cu & hi, jnp.float32)
            acc = acc + rlo * clo + rhi * chi
        outv[s] = acc
        return carry

    lax.fori_loop(0, _BPW // _L, group, 0)
    pltpu.sync_copy(outv, out_hbm.at[pl.ds(base, _BPW)])


def kernel(z, drp_rows, drp_cols):
    zf = _pack(z.T).reshape(-1)
    mesh = plsc.VectorSubcoreMesh(core_axis_name="c", subcore_axis_name="s")
    f = pl.kernel(
        _body,
        out_type=jax.ShapeDtypeStruct((_B,), jnp.float32),
        mesh=mesh,
        compiler_params=pltpu.CompilerParams(
            needs_layout_passes=False, use_tc_tiling_on_sc=True),
        scratch_types=[
            pltpu.VMEM((_NCH, _CH), jnp.int32),
            pltpu.VMEM((_NCH, _CH), jnp.int32),
            pltpu.VMEM((2 * _DP, _NCH, _CH), jnp.int32),
            pltpu.VMEM((_DP, _BPW), jnp.uint32),
            pltpu.VMEM((_DP, _BPW), jnp.uint32),
            pltpu.VMEM((_BPW,), jnp.float32),
            pltpu.SemaphoreType.DMA,
        ],
    )
    return f(zf, drp_rows.astype(jnp.int32), drp_cols.astype(jnp.int32))
